# Initial kernel scaffold; baseline (speedup 1.0000x reference)
import jax
import jax.numpy as jnp
from jax.experimental import pallas as pl


def kernel(x, edge_index, attrs, W, b):
    raise NotImplementedError("write your pallas kernel here")


# speedup vs baseline: 88.9950x; 88.9950x over previous
import functools

import jax
import jax.numpy as jnp
from jax import lax
from jax.experimental import pallas as pl
from jax.experimental.pallas import tpu as pltpu
from jax.experimental.pallas import tpu_sc as plsc

N_NODES = 10000
N_EDGES = 320000
D_FEAT = 128

NC, NS, L = 2, 16, 16
NW = NC * NS
NPAD = 10240
WIN = 80
ROWS = N_EDGES // WIN
RPT = ROWS // NW
EPT = N_EDGES // NW
ZCH = NPAD // NS

_mesh = plsc.VectorSubcoreMesh(
    core_axis_name="c", subcore_axis_name="s", num_cores=NC, num_subcores=NS
)
_sc_params = pltpu.CompilerParams(needs_layout_passes=False)


def _zero_shared(zv, sh, sid):
    @pl.loop(0, ZCH // L)
    def _(i):
        zv[pl.ds(i * L, L)] = jnp.zeros((L,), jnp.float32)

    pltpu.sync_copy(zv, sh.at[pl.ds(sid * ZCH, ZCH)])


@functools.partial(
    pl.kernel,
    out_type=jax.ShapeDtypeStruct((NC, NPAD), jnp.float32),
    mesh=_mesh,
    scratch_types=[
        pltpu.VMEM((RPT, WIN), jnp.int32),
        pltpu.VMEM((RPT, WIN), jnp.float32),
        pltpu.VMEM((ZCH,), jnp.float32),
        pltpu.VMEM_SHARED((NPAD,), jnp.float32),
    ],
    compiler_params=_sc_params,
)
def _sc_deg(col_hbm, attr_hbm, deg_hbm, colv, attrv, zv, deg_sh):
    cid = lax.axis_index("c")
    sid = lax.axis_index("s")
    wid = cid * NS + sid

    _zero_shared(zv, deg_sh, sid)
    pltpu.sync_copy(col_hbm.at[wid], colv)
    pltpu.sync_copy(attr_hbm.at[wid], attrv)
    plsc.subcore_barrier()

    @pl.loop(0, RPT)
    def _(j):
        pltpu.sync_copy(attrv.at[j], deg_sh.at[colv.at[j]], add=True)

    plsc.subcore_barrier()

    @pl.when(sid == 0)
    def _():
        pltpu.sync_copy(deg_sh, deg_hbm.at[cid])


@functools.partial(
    pl.kernel,
    out_type=jax.ShapeDtypeStruct((NC, NPAD), jnp.float32),
    mesh=_mesh,
    scratch_types=[
        pltpu.VMEM((RPT, WIN), jnp.int32),
        pltpu.VMEM((EPT,), jnp.int32),
        pltpu.VMEM((EPT,), jnp.float32),
        pltpu.VMEM((RPT, WIN), jnp.float32),
        pltpu.VMEM((NPAD,), jnp.float32),
        pltpu.VMEM((ZCH,), jnp.float32),
        pltpu.VMEM_SHARED((NPAD,), jnp.float32),
    ],
    compiler_params=_sc_params,
)
def _sc_msg(col_hbm, row_hbm, attr_hbm, g_hbm, s_hbm, colv, rowv, attrv, msgv, gv, zv, s_sh):
    cid = lax.axis_index("c")
    sid = lax.axis_index("s")
    wid = cid * NS + sid

    _zero_shared(zv, s_sh, sid)
    pltpu.sync_copy(g_hbm, gv)
    pltpu.sync_copy(col_hbm.at[wid], colv)
    pltpu.sync_copy(row_hbm.at[pl.ds(wid * EPT, EPT)], rowv)
    pltpu.sync_copy(attr_hbm.at[pl.ds(wid * EPT, EPT)], attrv)

    @pl.loop(0, RPT)
    def _(j):
        @pl.loop(0, WIN // L)
        def _(k):
            idx = rowv[pl.ds(j * WIN + k * L, L)]
            vals = plsc.load_gather(gv, [idx])
            msgv[j, pl.ds(k * L, L)] = vals * attrv[pl.ds(j * WIN + k * L, L)]

    plsc.subcore_barrier()

    @pl.loop(0, RPT)
    def _(j):
        pltpu.sync_copy(msgv.at[j], s_sh.at[colv.at[j]], add=True)

    plsc.subcore_barrier()

    @pl.when(sid == 0)
    def _():
        pltpu.sync_copy(s_sh, s_hbm.at[cid])


def _mv_body(w_ref, x_ref, o_ref):
    o_ref[...] = lax.dot_general(
        w_ref[...], x_ref[...], (((1,), (1,)), ((), ())),
        preferred_element_type=jnp.float32,
    )


def _pre_body(h_ref, d0_ref, d1_ref, g_ref, dis_ref):
    deg = d0_ref[...] + d1_ref[...] + 1.0
    dis = 1.0 / jnp.sqrt(deg)
    dis_ref[...] = dis
    g_ref[...] = h_ref[...] * dis


def _post_body(s0_ref, s1_ref, dis_ref, g_ref, b_ref, o_ref):
    z = b_ref[0, 0] + dis_ref[...] * (s0_ref[...] + s1_ref[...] + g_ref[...])
    o_ref[...] = z * jnp.tanh(jax.nn.softplus(z))


_P2 = (NPAD // 128, 128)


def kernel(x, edge_index, attrs, W, b):
    row = edge_index[0].astype(jnp.int32)
    col = edge_index[1].astype(jnp.int32)
    col3d = col.reshape(NW, RPT, WIN)
    attr3d = attrs.reshape(NW, RPT, WIN)

    h = pl.pallas_call(
        _mv_body, out_shape=jax.ShapeDtypeStruct((1, N_NODES), jnp.float32)
    )(W, x)

    degp = _sc_deg(col3d, attr3d)

    hp = jnp.pad(h, ((0, 0), (0, NPAD - N_NODES))).reshape(_P2)
    g2d, dis2d = pl.pallas_call(
        _pre_body,
        out_shape=(
            jax.ShapeDtypeStruct(_P2, jnp.float32),
            jax.ShapeDtypeStruct(_P2, jnp.float32),
        ),
    )(hp, degp[0].reshape(_P2), degp[1].reshape(_P2))

    sp = _sc_msg(col3d, row, attrs, g2d.reshape(-1))

    out2d = pl.pallas_call(
        _post_body, out_shape=jax.ShapeDtypeStruct(_P2, jnp.float32)
    )(sp[0].reshape(_P2), sp[1].reshape(_P2), dis2d, g2d, b.reshape(1, 1))

    return out2d.reshape(1, NPAD)[:, :N_NODES]


# speedup vs baseline: 118.3656x; 1.3300x over previous
import functools

import jax
import jax.numpy as jnp
from jax import lax
from jax.experimental import pallas as pl
from jax.experimental.pallas import tpu as pltpu
from jax.experimental.pallas import tpu_sc as plsc

N_NODES = 10000
N_EDGES = 320000
D_FEAT = 128

NC, NS, L = 2, 16, 16
NW = NC * NS
NPAD = 10240
WIN = 80
ROWS = N_EDGES // WIN
RPT = ROWS // NW
EPT = N_EDGES // NW
ZCH = NPAD // NS

_mesh = plsc.VectorSubcoreMesh(
    core_axis_name="c", subcore_axis_name="s", num_cores=NC, num_subcores=NS
)
_sc_params = pltpu.CompilerParams(needs_layout_passes=False)


def _zero_shared(zv, sh, sid):
    @pl.loop(0, ZCH // L)
    def _(i):
        zv[pl.ds(i * L, L)] = jnp.zeros((L,), jnp.float32)

    pltpu.sync_copy(zv, sh.at[pl.ds(sid * ZCH, ZCH)])


@functools.partial(
    pl.kernel,
    out_type=jax.ShapeDtypeStruct((NC, NPAD), jnp.float32),
    mesh=_mesh,
    scratch_types=[
        pltpu.VMEM((RPT, WIN), jnp.int32),
        pltpu.VMEM((RPT, WIN), jnp.float32),
        pltpu.VMEM((ZCH,), jnp.float32),
        pltpu.VMEM_SHARED((NPAD,), jnp.float32),
        pltpu.SemaphoreType.DMA,
    ],
    compiler_params=_sc_params,
)
def _sc_deg(col_hbm, attr_hbm, deg_hbm, colv, attrv, zv, deg_sh, sem):
    cid = lax.axis_index("c")
    sid = lax.axis_index("s")
    wid = cid * NS + sid

    c1 = pltpu.async_copy(col_hbm.at[wid], colv, sem)
    c2 = pltpu.async_copy(attr_hbm.at[wid], attrv, sem)
    _zero_shared(zv, deg_sh, sid)
    c1.wait()
    c2.wait()
    plsc.subcore_barrier()

    @pl.loop(0, RPT)
    def _(j):
        pltpu.async_copy(attrv.at[j], deg_sh.at[colv.at[j]], sem, add=True)

    @pl.loop(0, RPT)
    def _(j):
        pltpu.make_async_copy(attrv.at[j], deg_sh.at[colv.at[j]], sem).wait()

    plsc.subcore_barrier()

    @pl.when(sid == 0)
    def _():
        pltpu.sync_copy(deg_sh, deg_hbm.at[cid])


@functools.partial(
    pl.kernel,
    out_type=jax.ShapeDtypeStruct((NC, NPAD), jnp.float32),
    mesh=_mesh,
    scratch_types=[
        pltpu.VMEM((RPT, WIN), jnp.int32),
        pltpu.VMEM((EPT,), jnp.int32),
        pltpu.VMEM((EPT,), jnp.float32),
        pltpu.VMEM((RPT, WIN), jnp.float32),
        pltpu.VMEM((NPAD,), jnp.float32),
        pltpu.VMEM((ZCH,), jnp.float32),
        pltpu.VMEM_SHARED((NPAD,), jnp.float32),
        pltpu.SemaphoreType.DMA,
    ],
    compiler_params=_sc_params,
)
def _sc_msg(col_hbm, row_hbm, attr_hbm, g_hbm, s_hbm, colv, rowv, attrv, msgv, gv, zv, s_sh, sem):
    cid = lax.axis_index("c")
    sid = lax.axis_index("s")
    wid = cid * NS + sid

    c1 = pltpu.async_copy(g_hbm, gv, sem)
    c2 = pltpu.async_copy(col_hbm.at[wid], colv, sem)
    c3 = pltpu.async_copy(row_hbm.at[pl.ds(wid * EPT, EPT)], rowv, sem)
    c4 = pltpu.async_copy(attr_hbm.at[pl.ds(wid * EPT, EPT)], attrv, sem)
    _zero_shared(zv, s_sh, sid)
    c1.wait()
    c2.wait()
    c3.wait()
    c4.wait()
    plsc.subcore_barrier()

    @pl.loop(0, RPT)
    def _(j):
        @pl.loop(0, WIN // L)
        def _(k):
            idx = rowv[pl.ds(j * WIN + k * L, L)]
            vals = plsc.load_gather(gv, [idx])
            msgv[j, pl.ds(k * L, L)] = vals * attrv[pl.ds(j * WIN + k * L, L)]

        pltpu.async_copy(msgv.at[j], s_sh.at[colv.at[j]], sem, add=True)

    @pl.loop(0, RPT)
    def _(j):
        pltpu.make_async_copy(msgv.at[j], s_sh.at[colv.at[j]], sem).wait()

    plsc.subcore_barrier()

    @pl.when(sid == 0)
    def _():
        pltpu.sync_copy(s_sh, s_hbm.at[cid])


def _mv_body(w_ref, x_ref, o_ref):
    o_ref[...] = lax.dot_general(
        w_ref[...], x_ref[...], (((1,), (1,)), ((), ())),
        preferred_element_type=jnp.float32,
    )


def _pre_body(h_ref, d0_ref, d1_ref, g_ref, dis_ref):
    deg = d0_ref[...] + d1_ref[...] + 1.0
    dis = 1.0 / jnp.sqrt(deg)
    dis_ref[...] = dis
    g_ref[...] = h_ref[...] * dis


def _post_body(s0_ref, s1_ref, dis_ref, g_ref, b_ref, o_ref):
    z = b_ref[0, 0] + dis_ref[...] * (s0_ref[...] + s1_ref[...] + g_ref[...])
    o_ref[...] = z * jnp.tanh(jax.nn.softplus(z))


_P2 = (NPAD // 128, 128)


def kernel(x, edge_index, attrs, W, b):
    row = edge_index[0].astype(jnp.int32)
    col = edge_index[1].astype(jnp.int32)
    col3d = col.reshape(NW, RPT, WIN)
    attr3d = attrs.reshape(NW, RPT, WIN)

    h = pl.pallas_call(
        _mv_body, out_shape=jax.ShapeDtypeStruct((1, N_NODES), jnp.float32)
    )(W, x)

    degp = _sc_deg(col3d, attr3d)

    hp = jnp.pad(h, ((0, 0), (0, NPAD - N_NODES))).reshape(_P2)
    g2d, dis2d = pl.pallas_call(
        _pre_body,
        out_shape=(
            jax.ShapeDtypeStruct(_P2, jnp.float32),
            jax.ShapeDtypeStruct(_P2, jnp.float32),
        ),
    )(hp, degp[0].reshape(_P2), degp[1].reshape(_P2))

    sp = _sc_msg(col3d, row, attrs, g2d.reshape(-1))

    out2d = pl.pallas_call(
        _post_body, out_shape=jax.ShapeDtypeStruct(_P2, jnp.float32)
    )(sp[0].reshape(_P2), sp[1].reshape(_P2), dis2d, g2d, b.reshape(1, 1))

    return out2d.reshape(1, NPAD)[:, :N_NODES]
